# baseline (speedup 1.0000x reference)
import functools

import jax
import jax.numpy as jnp
from jax import lax
from jax.experimental import pallas as pl
from jax.experimental.pallas import tpu as pltpu
from jax.experimental.pallas import tpu_sc as plsc

BATCH = 4096
SEQ = 200
EMBED = 64
HIDDEN = 128
NUM_OUT = 3

NC = 2
NS = 16
NW = NC * NS
BPW = BATCH // NW
C0, C1 = 120, 80
LANES = 16
NVEC = EMBED // LANES


def _sc_gather_pool(flat_ids, embedding):
    mesh = plsc.VectorSubcoreMesh(core_axis_name="c", subcore_axis_name="s")

    @functools.partial(
        pl.kernel,
        out_type=jax.ShapeDtypeStruct((BATCH, EMBED), jnp.float32),
        mesh=mesh,
        compiler_params=pltpu.CompilerParams(use_tc_tiling_on_sc=False),
        scratch_types=[
            pltpu.VMEM((BPW * SEQ,), jnp.int32),
            pltpu.VMEM((C0, EMBED), jnp.float32),
            pltpu.VMEM((C1, EMBED), jnp.float32),
            pltpu.VMEM((BPW, EMBED), jnp.float32),
            pltpu.SemaphoreType.DMA,
        ],
    )
    def k(ids_hbm, table_hbm, out_hbm, idx_v, rows0_v, rows1_v, pooled_v, sem):
        wid = lax.axis_index("s") * NC + lax.axis_index("c")
        base = wid * BPW
        pltpu.sync_copy(ids_hbm.at[pl.ds(base * SEQ, BPW * SEQ)], idx_v)

        @pl.loop(0, BPW)
        def _(b):
            off = b * SEQ
            pltpu.async_copy(
                table_hbm.at[idx_v.at[pl.ds(off, C0)]], rows0_v, sem
            ).wait()
            pltpu.async_copy(
                table_hbm.at[idx_v.at[pl.ds(off + C0, C1)]], rows1_v, sem
            ).wait()

            def body0(r, acc):
                return tuple(
                    acc[d] + rows0_v[r, pl.ds(d * LANES, LANES)]
                    for d in range(NVEC)
                )

            acc = lax.fori_loop(
                0, C0, body0,
                tuple(jnp.zeros((LANES,), jnp.float32) for _ in range(NVEC)),
            )

            def body1(r, acc):
                return tuple(
                    acc[d] + rows1_v[r, pl.ds(d * LANES, LANES)]
                    for d in range(NVEC)
                )

            acc = lax.fori_loop(0, C1, body1, acc)

            for d in range(NVEC):
                pooled_v[b, pl.ds(d * LANES, LANES)] = acc[d]

        pltpu.sync_copy(pooled_v, out_hbm.at[pl.ds(base, BPW)])

    return k(flat_ids, embedding)


def _mlp(pooled, W1, b1, W2, b2):
    BB = 512

    def body(p_ref, w1_ref, b1_ref, w2_ref, b2_ref, o_ref):
        x = p_ref[...] * (1.0 / SEQ)
        h = jnp.dot(x, w1_ref[...], preferred_element_type=jnp.float32)
        h = jnp.maximum(h + b1_ref[...], 0.0)
        o_ref[...] = (
            jnp.dot(h, w2_ref[...], preferred_element_type=jnp.float32)
            + b2_ref[...]
        )

    return pl.pallas_call(
        body,
        grid=(BATCH // BB,),
        in_specs=[
            pl.BlockSpec((BB, EMBED), lambda i: (i, 0)),
            pl.BlockSpec((EMBED, HIDDEN), lambda i: (0, 0)),
            pl.BlockSpec((1, HIDDEN), lambda i: (0, 0)),
            pl.BlockSpec((HIDDEN, NUM_OUT), lambda i: (0, 0)),
            pl.BlockSpec((1, NUM_OUT), lambda i: (0, 0)),
        ],
        out_specs=pl.BlockSpec((BB, NUM_OUT), lambda i: (i, 0)),
        out_shape=jax.ShapeDtypeStruct((BATCH, NUM_OUT), jnp.float32),
    )(pooled, W1, b1.reshape(1, HIDDEN), W2, b2.reshape(1, NUM_OUT))


def kernel(input_ids, embedding, W1, b1, W2, b2):
    flat_ids = input_ids.reshape(-1).astype(jnp.int32)
    pooled = _sc_gather_pool(flat_ids, embedding)
    return _mlp(pooled, W1, b1, W2, b2)
